# baseline (speedup 1.0000x reference)
import functools

import jax
import jax.numpy as jnp
from jax import lax
from jax.experimental import pallas as pl
from jax.experimental.pallas import tpu as pltpu
from jax.experimental.pallas import tpu_sc as plsc

LAT = 128
NLAYER = 3
LANES = 16
_C = 80
_NCH = 126


def _sc_layer_fn(N, E):
    info = plsc.get_sparse_core_info()
    NC, NS = info.num_cores, info.num_subcores
    NW = NC * NS
    assert E == NW * _NCH * _C
    nzb = N // _C
    mesh = plsc.VectorSubcoreMesh(core_axis_name="c", subcore_axis_name="s")

    @functools.partial(
        pl.kernel,
        out_type=jax.ShapeDtypeStruct((NC, N, LAT), jnp.float32),
        mesh=mesh,
        scratch_types=[
            pltpu.VMEM((3, 2, _C), jnp.int32),
            pltpu.VMEM((3, _C), jnp.float32),
            pltpu.VMEM((3, _C), jnp.int32),
            pltpu.VMEM((_C, LAT), jnp.float32),
            pltpu.VMEM((_C, LAT), jnp.float32),
            pltpu.VMEM((_C, LAT), jnp.float32),
            pltpu.VMEM_SHARED((N, LAT), jnp.float32),
            pltpu.SemaphoreType.DMA,
            pltpu.SemaphoreType.DMA,
            pltpu.SemaphoreType.DMA,
            pltpu.SemaphoreType.DMA,
            pltpu.SemaphoreType.DMA,
            pltpu.SemaphoreType.DMA,
            pltpu.SemaphoreType.DMA,
            pltpu.SemaphoreType.DMA,
            pltpu.SemaphoreType.DMA,
        ],
    )
    def k(edges_h, w_h, x_h, out_h,
          ring, ringw, dcp, m0, m1, m2, acc,
          si0, si1, si2, sg0, sg1, sg2, ss0, ss1, ss2):
        cid = lax.axis_index("c")
        sid = lax.axis_index("s")
        wid = sid * NC + cid
        M = (m0, m1, m2)
        SI = (si0, si1, si2)
        SG = (sg0, sg1, sg2)
        SS = (ss0, ss1, ss2)

        def fire_edges(c, s):
            h = pltpu.async_copy(edges_h.at[wid, c], ring.at[s], SI[s])
            pltpu.async_copy(w_h.at[wid, c], ringw.at[s], SI[s])
            return h

        def edges_wait(c, s):
            pltpu.make_async_copy(edges_h.at[wid, c], ring.at[s], SI[s]).wait()
            pltpu.make_async_copy(w_h.at[wid, c], ringw.at[s], SI[s]).wait()

        def gather(c, s):
            pltpu.async_copy(x_h.at[ring.at[s, 0]], M[s], SG[s])

        def gather_wait(c, s):
            pltpu.make_async_copy(x_h.at[ring.at[s, 0]], M[s], SG[s]).wait()

        def scale_and_grab(s):
            def sgrp(g, _):
                lo = g * LANES
                wvec = ringw[s, pl.ds(lo, LANES)]
                dcp[s, pl.ds(lo, LANES)] = ring[s, 1, pl.ds(lo, LANES)]
                for t in range(LANES):
                    we = wvec[t]
                    e = lo + t
                    for j in range(LAT // LANES):
                        M[s][e, pl.ds(LANES * j, LANES)] = (
                            M[s][e, pl.ds(LANES * j, LANES)] * we)
                return _
            lax.fori_loop(0, _C // LANES, sgrp, None)

        def scatter(s):
            pltpu.async_copy(M[s], acc.at[dcp.at[s]], SS[s], add=True)

        def scatter_wait(s):
            pltpu.make_async_copy(M[s], acc.at[dcp.at[s]], SS[s]).wait()

        h0 = fire_edges(0, 0)
        h0b = pltpu.make_async_copy(w_h.at[wid, 0], ringw.at[0], SI[0])
        fire_edges(1, 1)

        for m in M:
            def zfill(r, _, m=m):
                for j in range(LAT // LANES):
                    m[r, pl.ds(LANES * j, LANES)] = jnp.zeros(
                        (LANES,), jnp.float32)
                return _
            lax.fori_loop(0, _C, zfill, None)

        def dzfill(r, _):
            for s in range(3):
                dcp[s, pl.ds(r * LANES, LANES)] = jnp.zeros((LANES,), jnp.int32)
            return _
        lax.fori_loop(0, _C // LANES, dzfill, None)

        def zcopy(t, _):
            blk = t * NS + sid
            @pl.when(blk < nzb)
            def _do():
                pltpu.sync_copy(m0, acc.at[pl.ds(blk * _C, _C)])
            return _
        lax.fori_loop(0, pl.cdiv(nzb, NS), zcopy, None)

        h0.wait()
        h0b.wait()
        gather(0, 0)
        plsc.subcore_barrier()
        scatter(1)
        scatter(2)

        def body3(g, _):
            for dc in range(3):
                c = 3 * g + dc
                q = dc
                qn = (dc + 1) % 3
                q2 = (dc + 2) % 3
                scatter_wait(qn)
                @pl.when(c + 2 < _NCH)
                def _fire():
                    fire_edges(c + 2, q2)
                @pl.when(c + 1 < _NCH)
                def _pref():
                    edges_wait(c + 1, qn)
                    gather(c + 1, qn)
                gather_wait(c, q)
                scale_and_grab(q)
                scatter(q)
            return _
        lax.fori_loop(0, _NCH // 3, body3, None)

        scatter_wait((_NCH - 2) % 3)
        scatter_wait((_NCH - 1) % 3)
        plsc.subcore_barrier()

        def wb_loop(t, _):
            blk = t * NS + sid
            @pl.when(blk < nzb)
            def _do():
                pltpu.sync_copy(acc.at[pl.ds(blk * _C, _C)],
                                out_h.at[cid, pl.ds(blk * _C, _C)])
            return _
        lax.fori_loop(0, pl.cdiv(nzb, NS), wb_loop, None)

    return k


def _merge(p, runsum):
    N, _ = runsum.shape
    blk = 400

    def mk(p_ref, rs_ref, x_ref, rs2_ref):
        a = p_ref[0] + p_ref[1]
        x_ref[...] = a
        rs2_ref[...] = rs_ref[...] + a

    return pl.pallas_call(
        mk,
        grid=(N // blk,),
        in_specs=[
            pl.BlockSpec((2, blk, LAT), lambda i: (0, i, 0)),
            pl.BlockSpec((blk, LAT), lambda i: (i, 0)),
        ],
        out_specs=[
            pl.BlockSpec((blk, LAT), lambda i: (i, 0)),
            pl.BlockSpec((blk, LAT), lambda i: (i, 0)),
        ],
        out_shape=[jax.ShapeDtypeStruct((N, LAT), jnp.float32)] * 2,
    )(p, runsum)


def kernel(edge_index, edge_weight, ini_embeds):
    N = ini_embeds.shape[0]
    E = edge_weight.shape[0]
    info = plsc.get_sparse_core_info()
    NW = info.num_cores * info.num_subcores
    ept = _NCH * _C
    pad = NW * ept - E
    src = jnp.concatenate([edge_index[0], jnp.zeros((pad,), edge_index.dtype)])
    dst = jnp.concatenate([edge_index[1], jnp.zeros((pad,), edge_index.dtype)])
    w = jnp.concatenate([edge_weight, jnp.zeros((pad,), edge_weight.dtype)])
    w = w.reshape(NW, _NCH, _C)
    edges = jnp.stack(
        [src.reshape(NW, _NCH, _C), dst.reshape(NW, _NCH, _C)],
        axis=2)
    layer = _sc_layer_fn(N, NW * ept)
    x = ini_embeds
    runsum = ini_embeds
    for _ in range(NLAYER):
        part = layer(edges, w, x)
        x, runsum = _merge(part, runsum)
    half = N // 2
    return runsum[:half], runsum[half:]
